# Initial kernel scaffold; baseline (speedup 1.0000x reference)
import jax
import jax.numpy as jnp
from jax.experimental import pallas as pl


def kernel(x, edge_index_rel0, edge_index_rel1, W0, attn_l0, attn_r0, gat_bias0, W1, attn_l1, attn_r1, gat_bias1, acr_attn_weights, h_bias, loop_weight):
    raise NotImplementedError("write your pallas kernel here")


# speedup vs baseline: 12.3160x; 12.3160x over previous
import jax
import jax.numpy as jnp
from jax import lax
from jax.experimental import pallas as pl
from jax.experimental.pallas import tpu as pltpu
from jax.experimental.pallas import tpu_sc as plsc

N = 10000
E = 80000
DIN = 512
DOUT = 512
H = 8
DH = 64
NPAD = 10240


_TN1 = 400


def _mm_body(x_ref, w_ref, o_ref):
    o_ref[...] = jnp.dot(x_ref[...], w_ref[...],
                         preferred_element_type=jnp.float32)


def _fused_matmul(x, wcat):
    kcols = wcat.shape[1]
    return pl.pallas_call(
        _mm_body,
        grid=(N // _TN1,),
        in_specs=[
            pl.BlockSpec((_TN1, DIN), lambda i: (i, 0)),
            pl.BlockSpec((DIN, kcols), lambda i: (0, 0)),
        ],
        out_specs=pl.BlockSpec((_TN1, kcols), lambda i: (i, 0)),
        out_shape=jax.ShapeDtypeStruct((N, kcols), jnp.float32),
    )(x, wcat)


_EPT = E // 16
_NCHUNK = 39
_K = 128
_NH = NPAD // 2
_NS = _NH // 16
_WBS = 80


def _sc_edge_body(src0, dst0, src1, dst1, elt, ert, feat0, feat1, out,
                  ela, era, elb, erb, pda, pdb,
                  src_buf, dst_buf, dloc_buf, soff_buf, eea_buf, eeb_buf,
                  rows, srcr, dstr, dlocr, soffr, dstage, dinva, dinvb, wb,
                  acc, dpartsa, sem):
    cid = lax.axis_index("c")
    tid = lax.axis_index("s")
    nbase = cid * _NH
    n0 = tid * _NS
    zeros16 = jnp.zeros((16,), jnp.float32)

    for rel in range(2):
        srcv = src0 if rel == 0 else src1
        dstv = dst0 if rel == 0 else dst1
        feat = feat0 if rel == 0 else feat1

        def one_pair(q, _, srcv=srcv, dstv=dstv, feat=feat, rel=rel):
            ha = 2 * q
            hb = 2 * q + 1
            pltpu.sync_copy(elt.at[rel, ha], ela)
            pltpu.sync_copy(ert.at[rel, ha], era)
            pltpu.sync_copy(elt.at[rel, hb], elb)
            pltpu.sync_copy(ert.at[rel, hb], erb)

            def zero_pd(i, _):
                pda[pl.ds(i * 16, 16)] = zeros16
                pdb[pl.ds(i * 16, 16)] = zeros16
                return 0
            lax.fori_loop(0, _NH // 16, zero_pd, 0)

            def zero_wb(k, _):
                for j in range(8):
                    wb[k, pl.ds(j * 16, 16)] = zeros16
                return 0
            lax.fori_loop(0, _WBS, zero_wb, 0)

            def zero_acc(ss, _):
                pltpu.sync_copy(wb, acc.at[pl.ds(n0 + ss * _WBS, _WBS)])
                return 0
            lax.fori_loop(0, _NS // _WBS, zero_acc, 0)
            plsc.subcore_barrier()

            qoff = q * N

            def score16(s16, d16):
                dloc = jnp.clip(d16 - nbase, 0, _NH - 1)
                inr = jnp.logical_and(d16 >= nbase, d16 < nbase + _NH)
                elsa = plsc.load_gather(ela, [s16])
                erda = plsc.load_gather(era, [d16])
                sa = elsa + erda
                sa = jnp.where(sa >= 0.0, sa, sa * 0.2)
                eea = jnp.where(inr, jnp.exp(sa), 0.0)
                elsb = plsc.load_gather(elb, [s16])
                erdb = plsc.load_gather(erb, [d16])
                sb = elsb + erdb
                sb = jnp.where(sb >= 0.0, sb, sb * 0.2)
                eeb = jnp.where(inr, jnp.exp(sb), 0.0)
                return dloc, eea, eeb

            def chunk(c, _):
                base = tid * _EPT + c * _K
                pltpu.sync_copy(srcv.at[pl.ds(base, _K)], src_buf)
                pltpu.sync_copy(dstv.at[pl.ds(base, _K)], dst_buf)
                for g in range(8):
                    s16 = src_buf[pl.ds(g * 16, 16)]
                    d16 = dst_buf[pl.ds(g * 16, 16)]
                    dloc, eea, eeb = score16(s16, d16)
                    eea_buf[pl.ds(g * 16, 16)] = eea
                    eeb_buf[pl.ds(g * 16, 16)] = eeb
                    dloc_buf[pl.ds(g * 16, 16)] = dloc
                    plsc.addupdate_scatter(pda, [dloc], eea)
                    plsc.addupdate_scatter(pdb, [dloc], eeb)
                    soff_buf[pl.ds(g * 16, 16)] = s16 + qoff
                pltpu.async_copy(feat.at[soff_buf], rows, sem).wait()

                def scale(g, _):
                    ea16 = eea_buf[pl.ds(g * 16, 16)]
                    eb16 = eeb_buf[pl.ds(g * 16, 16)]
                    for l in range(16):
                        a = ea16[l]
                        b = eb16[l]
                        k = g * 16 + l
                        for j in range(4):
                            rows[k, pl.ds(j * 16, 16)] = (
                                rows[k, pl.ds(j * 16, 16)] * a)
                        for j in range(4, 8):
                            rows[k, pl.ds(j * 16, 16)] = (
                                rows[k, pl.ds(j * 16, 16)] * b)
                    return 0
                lax.fori_loop(0, _K // 16, scale, 0)
                pltpu.sync_copy(rows, acc.at[dloc_buf], add=True)
                return 0
            lax.fori_loop(0, _NCHUNK, chunk, 0)

            rbase = tid * _EPT + _EPT - 16
            pltpu.sync_copy(srcv.at[pl.ds(rbase, 16)], srcr)
            pltpu.sync_copy(dstv.at[pl.ds(rbase, 16)], dstr)
            s16 = srcr[...]
            d16 = dstr[...]
            newm = lax.iota(jnp.int32, 16) >= 8
            dloc, eea, eeb = score16(s16, d16)
            eea = jnp.where(newm, eea, 0.0)
            eeb = jnp.where(newm, eeb, 0.0)
            dlocr[...] = dloc
            plsc.addupdate_scatter(pda, [dloc], eea, mask=newm)
            plsc.addupdate_scatter(pdb, [dloc], eeb, mask=newm)
            soffr[...] = s16 + qoff
            pltpu.async_copy(feat.at[soffr], rows.at[pl.ds(0, 16)], sem).wait()
            for l in range(16):
                a = eea[l]
                b = eeb[l]
                for j in range(4):
                    rows[l, pl.ds(j * 16, 16)] = rows[l, pl.ds(j * 16, 16)] * a
                for j in range(4, 8):
                    rows[l, pl.ds(j * 16, 16)] = rows[l, pl.ds(j * 16, 16)] * b
            pltpu.sync_copy(rows.at[pl.ds(0, 16)], acc.at[dlocr], add=True)
            plsc.subcore_barrier()

            def reduce_denom(pd, dinv):
                pltpu.sync_copy(pd, dpartsa.at[pl.ds(tid * _NH, _NH)])
                plsc.subcore_barrier()
                for p in range(16):
                    pltpu.sync_copy(dpartsa.at[pl.ds(p * _NH + n0, _NS)],
                                    dstage.at[pl.ds(p * _NS, _NS)])

                def red(j, _):
                    v = dstage[pl.ds(j * 16, 16)]
                    for p in range(1, 16):
                        v = v + dstage[pl.ds(p * _NS + j * 16, 16)]
                    v = jnp.where(v > 0.0, v, 1.0)
                    dinv[pl.ds(j * 16, 16)] = 1.0 / v
                    return 0
                lax.fori_loop(0, _NS // 16, red, 0)
                plsc.subcore_barrier()
            reduce_denom(pda, dinva)
            reduce_denom(pdb, dinvb)

            def writeback(ss, _):
                pltpu.sync_copy(acc.at[pl.ds(n0 + ss * _WBS, _WBS)], wb)

                def divide(g, _):
                    a16 = dinva[pl.ds(ss * _WBS + g * 16, 16)]
                    b16 = dinvb[pl.ds(ss * _WBS + g * 16, 16)]
                    for l in range(16):
                        a = a16[l]
                        b = b16[l]
                        k = g * 16 + l
                        for j in range(4):
                            wb[k, pl.ds(j * 16, 16)] = (
                                wb[k, pl.ds(j * 16, 16)] * a)
                        for j in range(4, 8):
                            wb[k, pl.ds(j * 16, 16)] = (
                                wb[k, pl.ds(j * 16, 16)] * b)
                    return 0
                lax.fori_loop(0, _WBS // 16, divide, 0)

                gn0 = nbase + n0 + ss * _WBS

                @pl.when(gn0 < N)
                def _():
                    pltpu.sync_copy(wb, out.at[rel, q, pl.ds(gn0, _WBS)])
                return 0
            lax.fori_loop(0, _NS // _WBS, writeback, 0)
            plsc.subcore_barrier()
            return 0

        lax.fori_loop(0, 4, one_pair, 0)


def _sc_edge(src0, dst0, src1, dst1, elt, ert, feat0, feat1):
    mesh = plsc.VectorSubcoreMesh(core_axis_name="c", subcore_axis_name="s")
    f32 = jnp.float32
    i32 = jnp.int32
    return pl.kernel(
        _sc_edge_body,
        out_type=jax.ShapeDtypeStruct((2, H // 2, N, 2 * DH), f32),
        mesh=mesh,
        scratch_types=[
            pltpu.VMEM((N,), f32),
            pltpu.VMEM((N,), f32),
            pltpu.VMEM((N,), f32),
            pltpu.VMEM((N,), f32),
            pltpu.VMEM((_NH,), f32),
            pltpu.VMEM((_NH,), f32),
            pltpu.VMEM((_K,), i32),
            pltpu.VMEM((_K,), i32),
            pltpu.VMEM((_K,), i32),
            pltpu.VMEM((_K,), i32),
            pltpu.VMEM((_K,), f32),
            pltpu.VMEM((_K,), f32),
            pltpu.VMEM((_K, 2 * DH), f32),
            pltpu.VMEM((16,), i32),
            pltpu.VMEM((16,), i32),
            pltpu.VMEM((16,), i32),
            pltpu.VMEM((16,), i32),
            pltpu.VMEM((16 * _NS,), f32),
            pltpu.VMEM((_NS,), f32),
            pltpu.VMEM((_NS,), f32),
            pltpu.VMEM((_WBS, 2 * DH), f32),
            pltpu.VMEM_SHARED((_NH, 2 * DH), f32),
            pltpu.VMEM_SHARED((16 * _NH,), f32),
            pltpu.SemaphoreType.DMA,
        ],
        compiler_params=pltpu.CompilerParams(needs_layout_passes=False),
    )(src0, dst0, src1, dst1, elt, ert, feat0, feat1)


_TN3 = 400


def _combine_body(a0_ref, a1_ref, zl_ref, gb0_ref, gb1_ref, aw_ref, hb_ref,
                  o_ref):
    g0 = a0_ref[...] + gb0_ref[...]
    g0 = jnp.where(g0 >= 0.0, g0, g0 * 0.2)
    g1 = a1_ref[...] + gb1_ref[...]
    g1 = jnp.where(g1 >= 0.0, g1, g1 * 0.2)
    aw = aw_ref[...]
    t0 = jnp.sum(g0 * aw, axis=-1, keepdims=True)
    t1 = jnp.sum(g1 * aw, axis=-1, keepdims=True)
    m = jnp.maximum(t0, t1)
    e0 = jnp.exp(t0 - m)
    e1 = jnp.exp(t1 - m)
    inv = 1.0 / (e0 + e1)
    o_ref[...] = (g0 * (e0 * inv) + g1 * (e1 * inv) + zl_ref[...]
                  + hb_ref[...])


def _combine(a0, a1, zloop, gb0, gb1, aw, hb):
    row = lambda i: (i, 0)
    fixed = lambda i: (0, 0)
    return pl.pallas_call(
        _combine_body,
        grid=(N // _TN3,),
        in_specs=[
            pl.BlockSpec((_TN3, DOUT), row),
            pl.BlockSpec((_TN3, DOUT), row),
            pl.BlockSpec((_TN3, DOUT), row),
            pl.BlockSpec((1, DOUT), fixed),
            pl.BlockSpec((1, DOUT), fixed),
            pl.BlockSpec((1, DOUT), fixed),
            pl.BlockSpec((1, DOUT), fixed),
        ],
        out_specs=pl.BlockSpec((_TN3, DOUT), row),
        out_shape=jax.ShapeDtypeStruct((N, DOUT), jnp.float32),
    )(a0, a1, zloop, gb0, gb1, aw, hb)


def kernel(x, edge_index_rel0, edge_index_rel1, W0, attn_l0, attn_r0,
           gat_bias0, W1, attn_l1, attn_r1, gat_bias1, acr_attn_weights,
           h_bias, loop_weight):
    f32 = jnp.float32
    wl0 = (W0.reshape(DIN, H, DH) * attn_l0[None]).sum(-1)
    wr0 = (W0.reshape(DIN, H, DH) * attn_r0[None]).sum(-1)
    wl1 = (W1.reshape(DIN, H, DH) * attn_l1[None]).sum(-1)
    wr1 = (W1.reshape(DIN, H, DH) * attn_r1[None]).sum(-1)
    wcat = jnp.concatenate(
        [W0, W1, loop_weight, wl0, wr0, wl1, wr1], axis=1).astype(f32)

    z = _fused_matmul(x.astype(f32), wcat)

    feat0 = z[:, :DOUT].reshape(N, H // 2, 2 * DH)
    feat0 = feat0.transpose(1, 0, 2).reshape(H // 2 * N, 2 * DH)
    feat1 = z[:, DOUT:2 * DOUT].reshape(N, H // 2, 2 * DH)
    feat1 = feat1.transpose(1, 0, 2).reshape(H // 2 * N, 2 * DH)
    zloop = z[:, 2 * DOUT:3 * DOUT]
    base = 3 * DOUT
    elt = jnp.stack([z[:, base:base + H].T, z[:, base + 2 * H:base + 3 * H].T])
    ert = jnp.stack([z[:, base + H:base + 2 * H].T,
                     z[:, base + 3 * H:base + 4 * H].T])

    ei0 = edge_index_rel0.astype(jnp.int32)
    ei1 = edge_index_rel1.astype(jnp.int32)
    agg = _sc_edge(ei0[0], ei0[1], ei1[0], ei1[1],
                   elt, ert, feat0, feat1)

    aggt = agg.transpose(0, 2, 1, 3).reshape(2, N, DOUT)
    return _combine(aggt[0], aggt[1], zloop,
                    gat_bias0.reshape(1, DOUT).astype(f32),
                    gat_bias1.reshape(1, DOUT).astype(f32),
                    acr_attn_weights.reshape(1, DOUT).astype(f32),
                    h_bias.reshape(1, DOUT).astype(f32))


# speedup vs baseline: 14.7061x; 1.1941x over previous
import jax
import jax.numpy as jnp
from jax import lax
from jax.experimental import pallas as pl
from jax.experimental.pallas import tpu as pltpu
from jax.experimental.pallas import tpu_sc as plsc

N = 10000
E = 80000
DIN = 512
DOUT = 512
H = 8
DH = 64
NPAD = 10240


_TN1 = 400


def _mm_body(x_ref, w_ref, o_ref):
    o_ref[...] = jnp.dot(x_ref[...], w_ref[...],
                         preferred_element_type=jnp.float32)


def _fused_matmul(x, wcat):
    kcols = wcat.shape[1]
    return pl.pallas_call(
        _mm_body,
        grid=(N // _TN1,),
        in_specs=[
            pl.BlockSpec((_TN1, DIN), lambda i: (i, 0)),
            pl.BlockSpec((DIN, kcols), lambda i: (0, 0)),
        ],
        out_specs=pl.BlockSpec((_TN1, kcols), lambda i: (i, 0)),
        out_shape=jax.ShapeDtypeStruct((N, kcols), jnp.float32),
    )(x, wcat)


_EPT = E // 16
_NCHUNK = 156
_K = 32
_NH = NPAD // 2
_NS = _NH // 16
_WBS = 80


def _sc_edge_body(src0, dst0, src1, dst1, elt, ert, feat0, feat1, out,
                  ela, era, elb, erb, pda, pdb, src_all, dst_all,
                  dlocA, soffA, eeaA, eebA, dlocB, soffB, eeaB, eebB,
                  rowsA, rowsB, dlocr, soffr, dstage, dinva, dinvb, wb,
                  acc, dpartsa, semA, semB):
    cid = lax.axis_index("c")
    tid = lax.axis_index("s")
    nbase = cid * _NH
    n0 = tid * _NS
    zeros16 = jnp.zeros((16,), jnp.float32)

    for rel in range(2):
        srcv = src0 if rel == 0 else src1
        dstv = dst0 if rel == 0 else dst1
        feat = feat0 if rel == 0 else feat1

        def one_pair(q, _, srcv=srcv, dstv=dstv, feat=feat, rel=rel):
            ha = 2 * q
            hb = 2 * q + 1
            pltpu.sync_copy(elt.at[rel, ha], ela)
            pltpu.sync_copy(ert.at[rel, ha], era)
            pltpu.sync_copy(elt.at[rel, hb], elb)
            pltpu.sync_copy(ert.at[rel, hb], erb)

            def zero_pd(i, _):
                pda[pl.ds(i * 16, 16)] = zeros16
                pdb[pl.ds(i * 16, 16)] = zeros16
                return 0
            lax.fori_loop(0, _NH // 16, zero_pd, 0)

            def zero_wb(k, _):
                for j in range(8):
                    wb[k, pl.ds(j * 16, 16)] = zeros16
                return 0
            lax.fori_loop(0, _WBS, zero_wb, 0)

            def zero_acc(ss, _):
                pltpu.sync_copy(wb, acc.at[pl.ds(n0 + ss * _WBS, _WBS)])
                return 0
            lax.fori_loop(0, _NS // _WBS, zero_acc, 0)
            plsc.subcore_barrier()

            qoff = q * N
            def load_idx(i, _):
                pltpu.sync_copy(srcv.at[pl.ds(tid * _EPT + i * 1000, 1000)],
                                src_all.at[pl.ds(i * 1000, 1000)])
                pltpu.sync_copy(dstv.at[pl.ds(tid * _EPT + i * 1000, 1000)],
                                dst_all.at[pl.ds(i * 1000, 1000)])
                return 0
            lax.fori_loop(0, _EPT // 1000, load_idx, 0)

            def score16(s16, d16):
                dloc = jnp.clip(d16 - nbase, 0, _NH - 1)
                inr = jnp.logical_and(d16 >= nbase, d16 < nbase + _NH)
                elsa = plsc.load_gather(ela, [s16])
                erda = plsc.load_gather(era, [d16])
                sa = elsa + erda
                sa = jnp.where(sa >= 0.0, sa, sa * 0.2)
                eea = jnp.where(inr, jnp.exp(sa), 0.0)
                elsb = plsc.load_gather(elb, [s16])
                erdb = plsc.load_gather(erb, [d16])
                sb = elsb + erdb
                sb = jnp.where(sb >= 0.0, sb, sb * 0.2)
                eeb = jnp.where(inr, jnp.exp(sb), 0.0)
                return dloc, eea, eeb

            def score_chunk(c, dlocb, soffb, eeab, eebb):
                for g in range(_K // 16):
                    off = c * _K + g * 16
                    s16 = src_all[pl.ds(off, 16)]
                    d16 = dst_all[pl.ds(off, 16)]
                    dloc, eea, eeb = score16(s16, d16)
                    eeab[pl.ds(g * 16, 16)] = eea
                    eebb[pl.ds(g * 16, 16)] = eeb
                    dlocb[pl.ds(g * 16, 16)] = dloc
                    plsc.addupdate_scatter(pda, [dloc], eea)
                    plsc.addupdate_scatter(pdb, [dloc], eeb)
                    soffb[pl.ds(g * 16, 16)] = s16 + qoff

            def scale_scatter(rowsb, dlocb, eeab, eebb):
                def scale(g, _):
                    ea16 = eeab[pl.ds(g * 16, 16)]
                    eb16 = eebb[pl.ds(g * 16, 16)]
                    for l in range(16):
                        a = ea16[l]
                        b = eb16[l]
                        k = g * 16 + l
                        for j in range(4):
                            rowsb[k, pl.ds(j * 16, 16)] = (
                                rowsb[k, pl.ds(j * 16, 16)] * a)
                        for j in range(4, 8):
                            rowsb[k, pl.ds(j * 16, 16)] = (
                                rowsb[k, pl.ds(j * 16, 16)] * b)
                    return 0
                lax.fori_loop(0, _K // 16, scale, 0)
                pltpu.sync_copy(rowsb, acc.at[dlocb], add=True)

            def pipe(cc, _):
                c0 = 2 * cc
                score_chunk(c0, dlocA, soffA, eeaA, eebA)
                pltpu.make_async_copy(feat.at[soffA], rowsA, semA).start()

                @pl.when(cc > 0)
                def _():
                    pltpu.make_async_copy(feat.at[soffB], rowsB, semB).wait()
                    scale_scatter(rowsB, dlocB, eeaB, eebB)

                score_chunk(c0 + 1, dlocB, soffB, eeaB, eebB)
                pltpu.make_async_copy(feat.at[soffB], rowsB, semB).start()
                pltpu.make_async_copy(feat.at[soffA], rowsA, semA).wait()
                scale_scatter(rowsA, dlocA, eeaA, eebA)
                return 0
            lax.fori_loop(0, _NCHUNK // 2, pipe, 0)
            pltpu.make_async_copy(feat.at[soffB], rowsB, semB).wait()
            scale_scatter(rowsB, dlocB, eeaB, eebB)

            s16 = src_all[pl.ds(_EPT - 16, 16)]
            d16 = dst_all[pl.ds(_EPT - 16, 16)]
            newm = lax.iota(jnp.int32, 16) >= 8
            dloc, eea, eeb = score16(s16, d16)
            eea = jnp.where(newm, eea, 0.0)
            eeb = jnp.where(newm, eeb, 0.0)
            dlocr[...] = dloc
            plsc.addupdate_scatter(pda, [dloc], eea, mask=newm)
            plsc.addupdate_scatter(pdb, [dloc], eeb, mask=newm)
            soffr[...] = s16 + qoff
            pltpu.async_copy(feat.at[soffr], rowsA.at[pl.ds(0, 16)],
                             semA).wait()
            for l in range(16):
                a = eea[l]
                b = eeb[l]
                for j in range(4):
                    rowsA[l, pl.ds(j * 16, 16)] = (
                        rowsA[l, pl.ds(j * 16, 16)] * a)
                for j in range(4, 8):
                    rowsA[l, pl.ds(j * 16, 16)] = (
                        rowsA[l, pl.ds(j * 16, 16)] * b)
            pltpu.sync_copy(rowsA.at[pl.ds(0, 16)], acc.at[dlocr], add=True)
            plsc.subcore_barrier()

            def reduce_denom(pd, dinv):
                dbase = cid * 16 * _NH
                pltpu.sync_copy(pd, dpartsa.at[pl.ds(dbase + tid * _NH, _NH)])
                plsc.subcore_barrier()
                for p in range(16):
                    pltpu.sync_copy(
                        dpartsa.at[pl.ds(dbase + p * _NH + n0, _NS)],
                        dstage.at[pl.ds(p * _NS, _NS)])

                def red(j, _):
                    v = dstage[pl.ds(j * 16, 16)]
                    for p in range(1, 16):
                        v = v + dstage[pl.ds(p * _NS + j * 16, 16)]
                    v = jnp.where(v > 0.0, v, 1.0)
                    dinv[pl.ds(j * 16, 16)] = 1.0 / v
                    return 0
                lax.fori_loop(0, _NS // 16, red, 0)
                plsc.subcore_barrier()
            reduce_denom(pda, dinva)
            reduce_denom(pdb, dinvb)

            def writeback(ss, _):
                pltpu.sync_copy(acc.at[pl.ds(n0 + ss * _WBS, _WBS)], wb)

                def divide(g, _):
                    a16 = dinva[pl.ds(ss * _WBS + g * 16, 16)]
                    b16 = dinvb[pl.ds(ss * _WBS + g * 16, 16)]
                    for l in range(16):
                        a = a16[l]
                        b = b16[l]
                        k = g * 16 + l
                        for j in range(4):
                            wb[k, pl.ds(j * 16, 16)] = (
                                wb[k, pl.ds(j * 16, 16)] * a)
                        for j in range(4, 8):
                            wb[k, pl.ds(j * 16, 16)] = (
                                wb[k, pl.ds(j * 16, 16)] * b)
                    return 0
                lax.fori_loop(0, _WBS // 16, divide, 0)

                gn0 = nbase + n0 + ss * _WBS

                @pl.when(gn0 < N)
                def _():
                    pltpu.sync_copy(wb, out.at[rel, q, pl.ds(gn0, _WBS)])
                return 0
            lax.fori_loop(0, _NS // _WBS, writeback, 0)
            plsc.subcore_barrier()
            return 0

        lax.fori_loop(0, 4, one_pair, 0)


def _sc_edge(src0, dst0, src1, dst1, elt, ert, feat0, feat1):
    mesh = plsc.VectorSubcoreMesh(core_axis_name="c", subcore_axis_name="s")
    f32 = jnp.float32
    i32 = jnp.int32
    return pl.kernel(
        _sc_edge_body,
        out_type=jax.ShapeDtypeStruct((2, H // 2, N, 2 * DH), f32),
        mesh=mesh,
        scratch_types=[
            pltpu.VMEM((N,), f32),
            pltpu.VMEM((N,), f32),
            pltpu.VMEM((N,), f32),
            pltpu.VMEM((N,), f32),
            pltpu.VMEM((_NH,), f32),
            pltpu.VMEM((_NH,), f32),
            pltpu.VMEM((_EPT,), i32),
            pltpu.VMEM((_EPT,), i32),
            pltpu.VMEM((_K,), i32),
            pltpu.VMEM((_K,), i32),
            pltpu.VMEM((_K,), f32),
            pltpu.VMEM((_K,), f32),
            pltpu.VMEM((_K,), i32),
            pltpu.VMEM((_K,), i32),
            pltpu.VMEM((_K,), f32),
            pltpu.VMEM((_K,), f32),
            pltpu.VMEM((_K, 2 * DH), f32),
            pltpu.VMEM((_K, 2 * DH), f32),
            pltpu.VMEM((16,), i32),
            pltpu.VMEM((16,), i32),
            pltpu.VMEM((16 * _NS,), f32),
            pltpu.VMEM((_NS,), f32),
            pltpu.VMEM((_NS,), f32),
            pltpu.VMEM((_WBS, 2 * DH), f32),
            pltpu.VMEM_SHARED((_NH, 2 * DH), f32),
            pltpu.HBM((2 * 16 * _NH,), f32),
            pltpu.SemaphoreType.DMA,
            pltpu.SemaphoreType.DMA,
        ],
        compiler_params=pltpu.CompilerParams(needs_layout_passes=False),
    )(src0, dst0, src1, dst1, elt, ert, feat0, feat1)


_TN3 = 400


def _combine_body(a0_ref, a1_ref, zl_ref, gb0_ref, gb1_ref, aw_ref, hb_ref,
                  o_ref):
    g0 = a0_ref[...] + gb0_ref[...]
    g0 = jnp.where(g0 >= 0.0, g0, g0 * 0.2)
    g1 = a1_ref[...] + gb1_ref[...]
    g1 = jnp.where(g1 >= 0.0, g1, g1 * 0.2)
    aw = aw_ref[...]
    t0 = jnp.sum(g0 * aw, axis=-1, keepdims=True)
    t1 = jnp.sum(g1 * aw, axis=-1, keepdims=True)
    m = jnp.maximum(t0, t1)
    e0 = jnp.exp(t0 - m)
    e1 = jnp.exp(t1 - m)
    inv = 1.0 / (e0 + e1)
    o_ref[...] = (g0 * (e0 * inv) + g1 * (e1 * inv) + zl_ref[...]
                  + hb_ref[...])


def _combine(a0, a1, zloop, gb0, gb1, aw, hb):
    row = lambda i: (i, 0)
    fixed = lambda i: (0, 0)
    return pl.pallas_call(
        _combine_body,
        grid=(N // _TN3,),
        in_specs=[
            pl.BlockSpec((_TN3, DOUT), row),
            pl.BlockSpec((_TN3, DOUT), row),
            pl.BlockSpec((_TN3, DOUT), row),
            pl.BlockSpec((1, DOUT), fixed),
            pl.BlockSpec((1, DOUT), fixed),
            pl.BlockSpec((1, DOUT), fixed),
            pl.BlockSpec((1, DOUT), fixed),
        ],
        out_specs=pl.BlockSpec((_TN3, DOUT), row),
        out_shape=jax.ShapeDtypeStruct((N, DOUT), jnp.float32),
    )(a0, a1, zloop, gb0, gb1, aw, hb)


def kernel(x, edge_index_rel0, edge_index_rel1, W0, attn_l0, attn_r0,
           gat_bias0, W1, attn_l1, attn_r1, gat_bias1, acr_attn_weights,
           h_bias, loop_weight):
    f32 = jnp.float32
    wl0 = (W0.reshape(DIN, H, DH) * attn_l0[None]).sum(-1)
    wr0 = (W0.reshape(DIN, H, DH) * attn_r0[None]).sum(-1)
    wl1 = (W1.reshape(DIN, H, DH) * attn_l1[None]).sum(-1)
    wr1 = (W1.reshape(DIN, H, DH) * attn_r1[None]).sum(-1)
    wcat = jnp.concatenate(
        [W0, W1, loop_weight, wl0, wr0, wl1, wr1], axis=1).astype(f32)

    z = _fused_matmul(x.astype(f32), wcat)

    feat0 = z[:, :DOUT].reshape(N, H // 2, 2 * DH)
    feat0 = feat0.transpose(1, 0, 2).reshape(H // 2 * N, 2 * DH)
    feat1 = z[:, DOUT:2 * DOUT].reshape(N, H // 2, 2 * DH)
    feat1 = feat1.transpose(1, 0, 2).reshape(H // 2 * N, 2 * DH)
    zloop = z[:, 2 * DOUT:3 * DOUT]
    base = 3 * DOUT
    elt = jnp.stack([z[:, base:base + H].T, z[:, base + 2 * H:base + 3 * H].T])
    ert = jnp.stack([z[:, base + H:base + 2 * H].T,
                     z[:, base + 3 * H:base + 4 * H].T])

    ei0 = edge_index_rel0.astype(jnp.int32)
    ei1 = edge_index_rel1.astype(jnp.int32)
    agg = _sc_edge(ei0[0], ei0[1], ei1[0], ei1[1],
                   elt, ert, feat0, feat1)

    aggt = agg.transpose(0, 2, 1, 3).reshape(2, N, DOUT)
    return _combine(aggt[0], aggt[1], zloop,
                    gat_bias0.reshape(1, DOUT).astype(f32),
                    gat_bias1.reshape(1, DOUT).astype(f32),
                    acr_attn_weights.reshape(1, DOUT).astype(f32),
                    h_bias.reshape(1, DOUT).astype(f32))
